# Initial kernel scaffold; baseline (speedup 1.0000x reference)
import jax
import jax.numpy as jnp
from jax.experimental import pallas as pl


def kernel(x, edge_index, W_in, W_hid, W_out):
    raise NotImplementedError("write your pallas kernel here")


# speedup vs baseline: 4.4014x; 4.4014x over previous
import functools

import jax
import jax.numpy as jnp
from jax import lax
from jax.experimental import pallas as pl
from jax.experimental.pallas import tpu as pltpu
from jax.experimental.pallas import tpu_sc as plsc

_NC = 2
_NS = 16


def _mm(x, W):
    N, D = x.shape
    Dout = W.shape[1]
    R = 1000
    grid = N // R

    def body(x_ref, w_ref, o_ref):
        o_ref[...] = jnp.dot(x_ref[...], w_ref[...],
                             preferred_element_type=jnp.float32)

    return pl.pallas_call(
        body,
        grid=(grid,),
        in_specs=[
            pl.BlockSpec((R, D), lambda i: (i, 0)),
            pl.BlockSpec((D, Dout), lambda i: (0, 0)),
        ],
        out_specs=pl.BlockSpec((R, Dout), lambda i: (i, 0)),
        out_shape=jax.ShapeDtypeStruct((N, Dout), jnp.float32),
    )(x, W)


def _fuse_mm(p, W):
    _, N, D = p.shape
    Dout = W.shape[1]
    R = 1000
    grid = N // R

    def body(p_ref, w_ref, o_ref):
        h = jnp.maximum(p_ref[0] + p_ref[1], 0.0)
        o_ref[...] = jnp.dot(h, w_ref[...],
                             preferred_element_type=jnp.float32)

    return pl.pallas_call(
        body,
        grid=(grid,),
        in_specs=[
            pl.BlockSpec((2, R, D), lambda i: (0, i, 0)),
            pl.BlockSpec((D, Dout), lambda i: (0, 0)),
        ],
        out_specs=pl.BlockSpec((R, Dout), lambda i: (i, 0)),
        out_shape=jax.ShapeDtypeStruct((N, Dout), jnp.float32),
    )(p, W)


def _relu_sum(p):
    _, N, D = p.shape
    R = 1000
    grid = N // R

    def body(p_ref, o_ref):
        o_ref[...] = jnp.maximum(p_ref[0] + p_ref[1], 0.0)

    return pl.pallas_call(
        body,
        grid=(grid,),
        in_specs=[pl.BlockSpec((2, R, D), lambda i: (0, i, 0))],
        out_specs=pl.BlockSpec((R, D), lambda i: (i, 0)),
        out_shape=jax.ShapeDtypeStruct((N, D), jnp.float32),
    )(p)


def _finish(p, W):
    _, N, D = p.shape
    Dout = W.shape[1]
    R = 1000
    grid = N // R

    def body(p_ref, w_ref, o_ref):
        z = jnp.dot(p_ref[0] + p_ref[1], w_ref[...],
                    preferred_element_type=jnp.float32)
        m = jnp.max(z, axis=1, keepdims=True)
        lse = jnp.log(jnp.sum(jnp.exp(z - m), axis=1, keepdims=True)) + m
        o_ref[...] = z - lse

    return pl.pallas_call(
        body,
        grid=(grid,),
        in_specs=[
            pl.BlockSpec((2, R, D), lambda i: (0, i, 0)),
            pl.BlockSpec((D, Dout), lambda i: (0, 0)),
        ],
        out_specs=pl.BlockSpec((R, Dout), lambda i: (i, 0)),
        out_shape=jax.ShapeDtypeStruct((N, Dout), jnp.float32),
    )(p, W)


def _sc_aggregate(m, src, dst, zeros):
    N, D = m.shape
    E = src.shape[0]
    NW = _NC * _NS
    per_w = E // NW
    C = 80
    steps = per_w // C
    rows_per_tile = (N // _NS) // 8 * 8
    tail_r0 = _NS * rows_per_tile
    tail_rows = N - tail_r0

    mesh = plsc.VectorSubcoreMesh(core_axis_name="c", subcore_axis_name="s")

    @functools.partial(
        pl.kernel,
        out_type=jax.ShapeDtypeStruct((_NC, N, D), jnp.float32),
        mesh=mesh,
        scratch_types=[
            pltpu.VMEM_SHARED((N, D), jnp.float32),
            pltpu.VMEM((C,), jnp.int32),
            pltpu.VMEM((C,), jnp.int32),
            pltpu.VMEM((C, D), jnp.float32),
            pltpu.SemaphoreType.DMA,
        ],
    )
    def body(m_hbm, src_hbm, dst_hbm, zeros_hbm, out_hbm,
             accum, src_v, dst_v, rows_v, sem):
        c = lax.axis_index("c")
        s = lax.axis_index("s")
        wid = c * _NS + s
        r0 = s * rows_per_tile

        pltpu.sync_copy(zeros_hbm.at[pl.ds(r0, rows_per_tile)],
                        accum.at[pl.ds(r0, rows_per_tile)])

        @pl.when(s == _NS - 1)
        def _():
            pltpu.sync_copy(zeros_hbm.at[pl.ds(tail_r0, tail_rows)],
                            accum.at[pl.ds(tail_r0, tail_rows)])

        plsc.subcore_barrier()

        base = wid * per_w

        def step(g, carry):
            off = pl.multiple_of(base + g * C, 8)
            pltpu.sync_copy(src_hbm.at[pl.ds(off, C)], src_v)
            pltpu.sync_copy(dst_hbm.at[pl.ds(off, C)], dst_v)
            pltpu.async_copy(m_hbm.at[src_v], rows_v, sem).wait()
            pltpu.sync_copy(rows_v, accum.at[dst_v], add=True)
            return carry

        lax.fori_loop(0, steps, step, 0)
        plsc.subcore_barrier()

        pltpu.sync_copy(accum.at[pl.ds(r0, rows_per_tile)],
                        out_hbm.at[c, pl.ds(r0, rows_per_tile)])

        @pl.when(s == _NS - 1)
        def _():
            pltpu.sync_copy(accum.at[pl.ds(tail_r0, tail_rows)],
                            out_hbm.at[c, pl.ds(tail_r0, tail_rows)])

    return body(m, src, dst, zeros)


def kernel(x, edge_index, W_in, W_hid, W_out):
    N = x.shape[0]
    src = edge_index[0].astype(jnp.int32)
    dst = edge_index[1].astype(jnp.int32)
    z_hid = jnp.zeros((N, W_in.shape[1]), jnp.float32)

    p = _sc_aggregate(_mm(x, W_in), src, dst, z_hid)
    p = _sc_aggregate(_fuse_mm(p, W_hid), src, dst, z_hid)
    p = _sc_aggregate(_relu_sum(p), src, dst, z_hid)
    return _finish(p, W_out)


# speedup vs baseline: 8.0097x; 1.8198x over previous
import functools

import jax
import jax.numpy as jnp
from jax import lax
from jax.experimental import pallas as pl
from jax.experimental.pallas import tpu as pltpu
from jax.experimental.pallas import tpu_sc as plsc

_NC = 2
_NS = 16


def _mm(x, W):
    N, D = x.shape
    Dout = W.shape[1]
    R = 1000
    grid = N // R

    def body(x_ref, w_ref, o_ref):
        o_ref[...] = jnp.dot(x_ref[...], w_ref[...],
                             preferred_element_type=jnp.float32)

    return pl.pallas_call(
        body,
        grid=(grid,),
        in_specs=[
            pl.BlockSpec((R, D), lambda i: (i, 0)),
            pl.BlockSpec((D, Dout), lambda i: (0, 0)),
        ],
        out_specs=pl.BlockSpec((R, Dout), lambda i: (i, 0)),
        out_shape=jax.ShapeDtypeStruct((N, Dout), jnp.float32),
    )(x, W)


def _fuse_mm(p, W):
    _, N, D = p.shape
    Dout = W.shape[1]
    R = 1000
    grid = N // R

    def body(p_ref, w_ref, o_ref):
        h = jnp.maximum(p_ref[0] + p_ref[1], 0.0)
        o_ref[...] = jnp.dot(h, w_ref[...],
                             preferred_element_type=jnp.float32)

    return pl.pallas_call(
        body,
        grid=(grid,),
        in_specs=[
            pl.BlockSpec((2, R, D), lambda i: (0, i, 0)),
            pl.BlockSpec((D, Dout), lambda i: (0, 0)),
        ],
        out_specs=pl.BlockSpec((R, Dout), lambda i: (i, 0)),
        out_shape=jax.ShapeDtypeStruct((N, Dout), jnp.float32),
    )(p, W)


def _relu_sum(p):
    _, N, D = p.shape
    R = 1000
    grid = N // R

    def body(p_ref, o_ref):
        o_ref[...] = jnp.maximum(p_ref[0] + p_ref[1], 0.0)

    return pl.pallas_call(
        body,
        grid=(grid,),
        in_specs=[pl.BlockSpec((2, R, D), lambda i: (0, i, 0))],
        out_specs=pl.BlockSpec((R, D), lambda i: (i, 0)),
        out_shape=jax.ShapeDtypeStruct((N, D), jnp.float32),
    )(p)


def _finish(p, W):
    _, N, D = p.shape
    Dout = W.shape[1]
    R = 1000
    grid = N // R

    def body(p_ref, w_ref, o_ref):
        z = jnp.dot(p_ref[0] + p_ref[1], w_ref[...],
                    preferred_element_type=jnp.float32)
        m = jnp.max(z, axis=1, keepdims=True)
        lse = jnp.log(jnp.sum(jnp.exp(z - m), axis=1, keepdims=True)) + m
        o_ref[...] = z - lse

    return pl.pallas_call(
        body,
        grid=(grid,),
        in_specs=[
            pl.BlockSpec((2, R, D), lambda i: (0, i, 0)),
            pl.BlockSpec((D, Dout), lambda i: (0, 0)),
        ],
        out_specs=pl.BlockSpec((R, Dout), lambda i: (i, 0)),
        out_shape=jax.ShapeDtypeStruct((N, Dout), jnp.float32),
    )(p, W)


_C = 80


def _sc_aggregate(m, src3, dst3, zeros):
    N, D = m.shape
    _, steps, C = dst3.shape
    per_w = steps * C
    rows_per_tile = (N // _NS) // 8 * 8
    tail_r0 = _NS * rows_per_tile
    tail_rows = N - tail_r0

    mesh = plsc.VectorSubcoreMesh(core_axis_name="c", subcore_axis_name="s")

    @functools.partial(
        pl.kernel,
        out_type=jax.ShapeDtypeStruct((_NC, N, D), jnp.float32),
        mesh=mesh,
        scratch_types=[
            pltpu.VMEM_SHARED((N, D), jnp.float32),
            pltpu.VMEM((per_w,), jnp.int32),
            pltpu.VMEM((steps, C), jnp.int32),
            [pltpu.VMEM((C, D), jnp.float32)] * 2,
            [pltpu.SemaphoreType.DMA] * 2,
            [pltpu.SemaphoreType.DMA] * 2,
        ],
    )
    def body(m_hbm, src_hbm, dst_hbm, zeros_hbm, out_hbm,
             accum, src_v, dst_v, rows_v, gsem, ssem):
        c = lax.axis_index("c")
        s = lax.axis_index("s")
        wid = c * _NS + s
        r0 = s * rows_per_tile

        pltpu.sync_copy(src_hbm.at[wid], src_v)
        pltpu.sync_copy(dst_hbm.at[wid], dst_v)

        pltpu.sync_copy(zeros_hbm.at[pl.ds(r0, rows_per_tile)],
                        accum.at[pl.ds(r0, rows_per_tile)])

        @pl.when(s == _NS - 1)
        def _():
            pltpu.sync_copy(zeros_hbm.at[pl.ds(tail_r0, tail_rows)],
                            accum.at[pl.ds(tail_r0, tail_rows)])

        plsc.subcore_barrier()

        def gather_start(g, j):
            pltpu.async_copy(m_hbm.at[src_v.at[pl.ds(g * C, C)]],
                             rows_v[j], gsem[j])

        def gather_wait(g, j):
            pltpu.make_async_copy(m_hbm.at[src_v.at[pl.ds(g * C, C)]],
                                  rows_v[j], gsem[j]).wait()

        def scatter_start(g, j):
            pltpu.async_copy(rows_v[j], accum.at[dst_v.at[g]], ssem[j],
                             add=True)

        def scatter_wait(j):
            pltpu.make_async_copy(rows_v[j], accum.at[pl.ds(0, C)],
                                  ssem[j]).wait()

        def phase(g, j, primed):
            if not primed:
                scatter_wait(j)
                gather_start(g, j)
            gather_wait(g, j)
            scatter_start(g, j)

        gather_start(0, 0)
        gather_start(1, 1)
        phase(0, 0, True)
        phase(1, 1, True)

        def step(i, carry):
            phase(2 * i + 2, 0, False)
            phase(2 * i + 3, 1, False)
            return carry

        lax.fori_loop(0, (steps - 2) // 2, step, 0)
        if steps % 2 == 1:
            phase(steps - 1, 0, False)
            scatter_wait(1)
            scatter_wait(0)
        else:
            scatter_wait(0)
            scatter_wait(1)
        plsc.subcore_barrier()

        pltpu.sync_copy(accum.at[pl.ds(r0, rows_per_tile)],
                        out_hbm.at[c, pl.ds(r0, rows_per_tile)])

        @pl.when(s == _NS - 1)
        def _():
            pltpu.sync_copy(accum.at[pl.ds(tail_r0, tail_rows)],
                            out_hbm.at[c, pl.ds(tail_r0, tail_rows)])

    return body(m, src3, dst3, zeros)


def kernel(x, edge_index, W_in, W_hid, W_out):
    N = x.shape[0]
    E = edge_index.shape[1]
    NW = _NC * _NS
    src = edge_index[0].astype(jnp.int32)
    dst = edge_index[1].astype(jnp.int32)

    per_w = E // NW
    steps = per_w // _C
    src3 = src.reshape(NW, per_w)
    dst3 = dst.reshape(NW, steps, _C)
    z_hid = jnp.zeros((N, W_in.shape[1]), jnp.float32)

    p = _sc_aggregate(_mm(x, W_in), src3, dst3, z_hid)
    p = _sc_aggregate(_fuse_mm(p, W_hid), src3, dst3, z_hid)
    p = _sc_aggregate(_relu_sum(p), src3, dst3, z_hid)
    return _finish(p, W_out)


# speedup vs baseline: 11.9985x; 1.4980x over previous
import functools

import jax
import jax.numpy as jnp
from jax import lax
from jax.experimental import pallas as pl
from jax.experimental.pallas import tpu as pltpu
from jax.experimental.pallas import tpu_sc as plsc

_NC = 2
_NS = 16


def _mm(x, W):
    N, D = x.shape
    Dout = W.shape[1]
    R = 1000
    grid = N // R

    def body(x_ref, w_ref, o_ref):
        o_ref[...] = jnp.dot(x_ref[...], w_ref[...],
                             preferred_element_type=jnp.float32)

    return pl.pallas_call(
        body,
        grid=(grid,),
        in_specs=[
            pl.BlockSpec((R, D), lambda i: (i, 0)),
            pl.BlockSpec((D, Dout), lambda i: (0, 0)),
        ],
        out_specs=pl.BlockSpec((R, Dout), lambda i: (i, 0)),
        out_shape=jax.ShapeDtypeStruct((N, Dout), jnp.float32),
    )(x, W)


def _fuse_mm(p, W):
    _, N, D = p.shape
    Dout = W.shape[1]
    R = 1000
    grid = N // R

    def body(p_ref, w_ref, o_ref):
        h = jnp.maximum(p_ref[0] + p_ref[1], 0.0)
        o_ref[...] = jnp.dot(h, w_ref[...],
                             preferred_element_type=jnp.float32)

    return pl.pallas_call(
        body,
        grid=(grid,),
        in_specs=[
            pl.BlockSpec((2, R, D), lambda i: (0, i, 0)),
            pl.BlockSpec((D, Dout), lambda i: (0, 0)),
        ],
        out_specs=pl.BlockSpec((R, Dout), lambda i: (i, 0)),
        out_shape=jax.ShapeDtypeStruct((N, Dout), jnp.float32),
    )(p, W)


def _relu_sum(p):
    _, N, D = p.shape
    R = 1000
    grid = N // R

    def body(p_ref, o_ref):
        o_ref[...] = jnp.maximum(p_ref[0] + p_ref[1], 0.0)

    return pl.pallas_call(
        body,
        grid=(grid,),
        in_specs=[pl.BlockSpec((2, R, D), lambda i: (0, i, 0))],
        out_specs=pl.BlockSpec((R, D), lambda i: (i, 0)),
        out_shape=jax.ShapeDtypeStruct((N, D), jnp.float32),
    )(p)


def _finish(p, W):
    _, N, D = p.shape
    Dout = W.shape[1]
    R = 1000
    grid = N // R

    def body(p_ref, w_ref, o_ref):
        z = jnp.dot(p_ref[0] + p_ref[1], w_ref[...],
                    preferred_element_type=jnp.float32)
        m = jnp.max(z, axis=1, keepdims=True)
        lse = jnp.log(jnp.sum(jnp.exp(z - m), axis=1, keepdims=True)) + m
        o_ref[...] = z - lse

    return pl.pallas_call(
        body,
        grid=(grid,),
        in_specs=[
            pl.BlockSpec((2, R, D), lambda i: (0, i, 0)),
            pl.BlockSpec((D, Dout), lambda i: (0, 0)),
        ],
        out_specs=pl.BlockSpec((R, Dout), lambda i: (i, 0)),
        out_shape=jax.ShapeDtypeStruct((N, Dout), jnp.float32),
    )(p, W)


_C = 80


def _sc_aggregate(m, pk3, steps, zeros):
    N, D = m.shape
    per_w = pk3.shape[1]
    C = _C
    rows_per_tile = (N // _NS) // 8 * 8
    tail_r0 = _NS * rows_per_tile
    tail_rows = N - tail_r0

    mesh = plsc.VectorSubcoreMesh(core_axis_name="c", subcore_axis_name="s")

    @functools.partial(
        pl.kernel,
        out_type=jax.ShapeDtypeStruct((_NC, N, D), jnp.float32),
        mesh=mesh,
        scratch_types=[
            pltpu.VMEM_SHARED((N, D), jnp.float32),
            pltpu.VMEM((per_w,), jnp.int32),
            [pltpu.VMEM((C,), jnp.int32)] * 3,
            [pltpu.VMEM((C,), jnp.int32)] * 3,
            [pltpu.VMEM((C, D), jnp.float32)] * 3,
            [pltpu.SemaphoreType.DMA] * 3,
            [pltpu.SemaphoreType.DMA] * 3,
        ],
    )
    def body(m_hbm, pk_hbm, zeros_hbm, out_hbm,
             accum, pk_v, src_st, dst_st, rows_v, gsem, ssem):
        c = lax.axis_index("c")
        s = lax.axis_index("s")
        wid = c * _NS + s
        r0 = s * rows_per_tile

        pltpu.sync_copy(pk_hbm.at[wid], pk_v)

        pltpu.sync_copy(zeros_hbm.at[pl.ds(r0, rows_per_tile)],
                        accum.at[pl.ds(r0, rows_per_tile)])

        @pl.when(s == _NS - 1)
        def _():
            pltpu.sync_copy(zeros_hbm.at[pl.ds(tail_r0, tail_rows)],
                            accum.at[pl.ds(tail_r0, tail_rows)])

        plsc.subcore_barrier()

        def unpack(g, j):
            for k in range(C // 16):
                v = pk_v[pl.ds(g * C + k * 16, 16)]
                src_st[j][pl.ds(k * 16, 16)] = v & 0x3FFF
                dst_st[j][pl.ds(k * 16, 16)] = v >> 14

        def gather_start(g, j):
            del g
            pltpu.async_copy(m_hbm.at[src_st[j]], rows_v[j], gsem[j])

        def gather_wait(j):
            pltpu.make_async_copy(m_hbm.at[src_st[j]], rows_v[j],
                                  gsem[j]).wait()

        def scatter_start(j):
            pltpu.async_copy(rows_v[j], accum.at[dst_st[j]], ssem[j],
                             add=True)

        def scatter_wait(j):
            pltpu.make_async_copy(rows_v[j], accum.at[pl.ds(0, C)],
                                  ssem[j]).wait()

        unpack(0, 0)
        unpack(1, 1)
        gather_start(0, 0)
        gather_start(1, 1)

        def phase(g, j):
            jp = (j + 2) % 3
            g = jnp.int32(g)

            @pl.when(g >= 1)
            def _():
                scatter_wait(jp)

            @pl.when(g + 2 < steps)
            def _():
                unpack(g + 2, jp)
                gather_start(g + 2, jp)

            gather_wait(j)
            scatter_start(j)

        def step(i, carry):
            phase(3 * i, 0)
            phase(3 * i + 1, 1)
            phase(3 * i + 2, 2)
            return carry

        full = steps // 3
        lax.fori_loop(0, full, step, 0)
        for g in range(full * 3, steps):
            phase(g, g % 3)
        scatter_wait((steps - 1) % 3)
        plsc.subcore_barrier()

        pltpu.sync_copy(accum.at[pl.ds(r0, rows_per_tile)],
                        out_hbm.at[c, pl.ds(r0, rows_per_tile)])

        @pl.when(s == _NS - 1)
        def _():
            pltpu.sync_copy(accum.at[pl.ds(tail_r0, tail_rows)],
                            out_hbm.at[c, pl.ds(tail_r0, tail_rows)])

    return body(m, pk3, zeros)


def kernel(x, edge_index, W_in, W_hid, W_out):
    N = x.shape[0]
    E = edge_index.shape[1]
    NW = _NC * _NS
    src = edge_index[0].astype(jnp.int32)
    dst = edge_index[1].astype(jnp.int32)

    per_w = E // NW
    steps = per_w // _C
    pk3 = jnp.bitwise_or(jnp.left_shift(dst, 14), src).reshape(NW, per_w)
    z_hid = jnp.zeros((N, W_in.shape[1]), jnp.float32)

    p = _sc_aggregate(_mm(x, W_in), pk3, steps, z_hid)
    p = _sc_aggregate(_fuse_mm(p, W_hid), pk3, steps, z_hid)
    p = _sc_aggregate(_relu_sum(p), pk3, steps, z_hid)
    return _finish(p, W_out)
